# baseline (speedup 1.0000x reference)
import jax
import jax.numpy as jnp
from jax.experimental import pallas as pl


def _body(x_ref, a_ref, m_ref, o_ref):
    m = m_ref[...]
    o_ref[...] = jnp.where(m != 0.0, x_ref[...] + a_ref[...], x_ref[...])


def kernel(x, attack, attack_mask):
    B, S, D = x.shape
    N = B * S
    xf = x.reshape(N, D)
    af = attack.reshape(N, D)
    mf = attack_mask[:, :S].reshape(N, 1).astype(jnp.float32)
    R = 256
    out = pl.pallas_call(
        _body,
        grid=(N // R,),
        in_specs=[
            pl.BlockSpec((R, D), lambda i: (i, 0)),
            pl.BlockSpec((R, D), lambda i: (i, 0)),
            pl.BlockSpec((R, 1), lambda i: (i, 0)),
        ],
        out_specs=pl.BlockSpec((R, D), lambda i: (i, 0)),
        out_shape=jax.ShapeDtypeStruct((N, D), x.dtype),
    )(xf, af, mf)
    return out.reshape(B, S, D)
